# Initial kernel scaffold; baseline (speedup 1.0000x reference)
import jax
import jax.numpy as jnp
from jax.experimental import pallas as pl


def kernel(x, edge_index, W0, b0, W1, b1, W2, b2, W3, b3):
    raise NotImplementedError("write your pallas kernel here")


# speedup vs baseline: 4.5989x; 4.5989x over previous
import functools

import jax
import jax.numpy as jnp
from jax import lax
from jax.experimental import pallas as pl
from jax.experimental.pallas import tpu as pltpu
from jax.experimental.pallas import tpu_sc as plsc

N_NODES = 10000
N_EDGES = 320000
NFEAT = 128
NCLASS = 40

NC = 2
NS = 16
NW = NC * NS

EDGES_PER_TILE = N_EDGES // NW
CHUNK = 80
NCHUNK = EDGES_PER_TILE // CHUNK
ROWS_PER_TILE = 640
ROW_CHUNK = 80


def _spmm_body(h_hbm, src_hbm, dst_hbm, out_hbm,
               src_v, dst_v, rows_v, sem, acc_sh):
    c = lax.axis_index("c")
    s = lax.axis_index("s")
    r0 = s * ROWS_PER_TILE
    n_row_chunks = jnp.where(s == NS - 1, 5, 8)

    def init_body(k, carry):
        off = r0 + k * ROW_CHUNK
        pltpu.sync_copy(h_hbm.at[pl.ds(off, ROW_CHUNK)],
                        acc_sh.at[pl.ds(off, ROW_CHUNK)])
        return carry
    lax.fori_loop(0, n_row_chunks, init_body, 0)
    plsc.subcore_barrier()

    ebase = (c * NS + s) * EDGES_PER_TILE

    def edge_body(k, carry):
        base = ebase + k * CHUNK
        pltpu.sync_copy(src_hbm.at[pl.ds(base, CHUNK)], src_v)
        pltpu.sync_copy(dst_hbm.at[pl.ds(base, CHUNK)], dst_v)
        pltpu.async_copy(h_hbm.at[src_v], rows_v, sem).wait()
        pltpu.sync_copy(rows_v, acc_sh.at[dst_v], add=True)
        return carry
    lax.fori_loop(0, NCHUNK, edge_body, 0)
    plsc.subcore_barrier()

    def out_body(k, carry):
        off = r0 + k * ROW_CHUNK
        pltpu.sync_copy(acc_sh.at[pl.ds(off, ROW_CHUNK)],
                        out_hbm.at[pl.ds(c * N_NODES + off, ROW_CHUNK)])
        return carry
    lax.fori_loop(0, n_row_chunks, out_body, 0)


@jax.jit
def _spmm(h, src, dst):
    mesh = plsc.VectorSubcoreMesh(core_axis_name="c", subcore_axis_name="s")
    return pl.kernel(
        _spmm_body,
        out_type=jax.ShapeDtypeStruct((NC * N_NODES, NFEAT), jnp.float32),
        mesh=mesh,
        scratch_types=[
            pltpu.VMEM((CHUNK,), jnp.int32),
            pltpu.VMEM((CHUNK,), jnp.int32),
            pltpu.VMEM((CHUNK, NFEAT), jnp.float32),
            pltpu.SemaphoreType.DMA,
            pltpu.VMEM_SHARED((N_NODES, NFEAT), jnp.float32),
        ],
    )(h, src, dst)


BR = 1000


def _mlp01_body(x_r, p0_r, p1_r, w0_r, b0_r, w1_r, b1_r, o_r):
    a = p0_r[...] + p1_r[...] - x_r[...]
    h = jnp.dot(a, w0_r[...], preferred_element_type=jnp.float32) + b0_r[...]
    h = jnp.maximum(h, 0.0)
    h = jnp.dot(h, w1_r[...], preferred_element_type=jnp.float32) + b1_r[...]
    o_r[...] = jnp.maximum(h, 0.0)


@jax.jit
def _mlp01(x, p, W0, b0, W1, b1):
    grid = (N_NODES // BR,)
    return pl.pallas_call(
        _mlp01_body,
        grid=grid,
        in_specs=[
            pl.BlockSpec((BR, NFEAT), lambda i: (i, 0)),
            pl.BlockSpec((BR, NFEAT), lambda i: (i, 0)),
            pl.BlockSpec((BR, NFEAT), lambda i: (i + N_NODES // BR, 0)),
            pl.BlockSpec((NFEAT, NFEAT), lambda i: (0, 0)),
            pl.BlockSpec((1, NFEAT), lambda i: (0, 0)),
            pl.BlockSpec((NFEAT, NFEAT), lambda i: (0, 0)),
            pl.BlockSpec((1, NFEAT), lambda i: (0, 0)),
        ],
        out_specs=pl.BlockSpec((BR, NFEAT), lambda i: (i, 0)),
        out_shape=jax.ShapeDtypeStruct((N_NODES, NFEAT), jnp.float32),
    )(x, p, p, W0, b0.reshape(1, NFEAT), W1, b1.reshape(1, NFEAT))


def _mlp23_body(h_r, q0_r, q1_r, w2_r, b2_r, w3_r, b3_r, o_r):
    a = q0_r[...] + q1_r[...] - h_r[...]
    h = jnp.dot(a, w2_r[...], preferred_element_type=jnp.float32) + b2_r[...]
    h = jnp.maximum(h, 0.0)
    logits = jnp.dot(h, w3_r[...], preferred_element_type=jnp.float32) + b3_r[...]
    m = jnp.max(logits, axis=1, keepdims=True)
    z = logits - m
    o_r[...] = z - jnp.log(jnp.sum(jnp.exp(z), axis=1, keepdims=True))


@jax.jit
def _mlp23(h, q, W2, b2, W3p, b3p):
    grid = (N_NODES // BR,)
    return pl.pallas_call(
        _mlp23_body,
        grid=grid,
        in_specs=[
            pl.BlockSpec((BR, NFEAT), lambda i: (i, 0)),
            pl.BlockSpec((BR, NFEAT), lambda i: (i, 0)),
            pl.BlockSpec((BR, NFEAT), lambda i: (i + N_NODES // BR, 0)),
            pl.BlockSpec((NFEAT, NFEAT), lambda i: (0, 0)),
            pl.BlockSpec((1, NFEAT), lambda i: (0, 0)),
            pl.BlockSpec((NFEAT, NFEAT), lambda i: (0, 0)),
            pl.BlockSpec((1, NFEAT), lambda i: (0, 0)),
        ],
        out_specs=pl.BlockSpec((BR, NFEAT), lambda i: (i, 0)),
        out_shape=jax.ShapeDtypeStruct((N_NODES, NFEAT), jnp.float32),
    )(h, q, q, W2, b2.reshape(1, NFEAT), W3p, b3p)


def kernel(x, edge_index, W0, b0, W1, b1, W2, b2, W3, b3):
    dst = edge_index[0]
    src = edge_index[1]

    p = _spmm(x, src, dst)
    h2 = _mlp01(x, p, W0, b0, W1, b1)
    q = _spmm(h2, src, dst)

    W3p = jnp.zeros((NFEAT, NFEAT), jnp.float32).at[:, :NCLASS].set(W3)
    b3p = jnp.full((1, NFEAT), -1e30, jnp.float32).at[0, :NCLASS].set(b3)
    out = _mlp23(h2, q, W2, b2, W3p, b3p)
    return out[:, :NCLASS]
